# baseline (speedup 1.0000x reference)
import functools

import jax
import jax.numpy as jnp
from jax import lax
from jax.experimental import pallas as pl
from jax.experimental.pallas import tpu as pltpu
from jax.experimental.pallas import tpu_sc as plsc

H = 512
W = 512
B = 8
N = 4096
HW = H * W

NC = 2
NS = 16
NW = NC * NS
SPW = (B * N) // NW
CHUNKS = SPW // 16
WPB = NW // B
QS = SPW // 4

_MESH = plsc.VectorSubcoreMesh(core_axis_name="c", subcore_axis_name="s")

_SCRATCH = [
    pltpu.VMEM((SPW * 3,), jnp.int32),
    pltpu.VMEM((SPW * 3,), jnp.float32),
    pltpu.VMEM((SPW, 16), jnp.float32),
    pltpu.VMEM((SPW,), jnp.int32),
    pltpu.SemaphoreType.DMA,
    pltpu.SemaphoreType.DMA,
]

_PARAMS = pltpu.CompilerParams(needs_layout_passes=False,
                               use_tc_tiling_on_sc=False)


def _gather_quarter(table_hbm, fidx_v, gat_v, gsem, q):
    return [
        pltpu.async_copy(
            table_hbm.at[fidx_v.at[pl.ds(c * SPW + q * QS, QS)]],
            gat_v.at[pl.ds(c * SPW + q * QS, QS)], gsem)
        for c in range(3)
    ]


def _pipeline(out_ref, gat_v, stage, rowidx_v, wsem, wid, gathers):
    iota = lax.iota(jnp.int32, 16)

    def fill_quarter(q):
        def body(k, carry):
            rows = q * QS + k * 16 + iota
            rowidx_v[pl.ds(q * QS + k * 16, 16)] = wid * SPW + rows
            for c in range(3):
                vals = gat_v[pl.ds(c * SPW + q * QS + k * 16, 16)]
                plsc.store_scatter(
                    stage, [rows, jnp.full((16,), c, jnp.int32)], vals)
            return carry
        lax.fori_loop(0, QS // 16, body, 0)

    for q in range(4):
        for cp in gathers[q]:
            cp.wait()
        fill_quarter(q)
    pltpu.async_copy(stage, out_ref.at[rowidx_v], wsem).wait()


@functools.partial(
    pl.kernel,
    mesh=_MESH,
    out_type=[
        jax.ShapeDtypeStruct((B * N, 16), jnp.float32),
        jax.ShapeDtypeStruct((B * N * 3,), jnp.int32),
    ],
    scratch_types=[pltpu.VMEM((SPW,), jnp.int32), *_SCRATCH,
                   pltpu.SemaphoreType.DMA],
    compiler_params=_PARAMS,
)
def _sample_dirs(idx_hbm, table_hbm, out_ref, fidx_out,
                 idx_v, fidx_v, gat_v, stage, rowidx_v, gsem, wsem, fsem):
    wid = lax.axis_index("s") * NC + lax.axis_index("c")
    b = wid // WPB
    pltpu.sync_copy(idx_hbm.at[pl.ds(wid * SPW, SPW)], idx_v)

    def build(lo, hi):
        def body(j, carry):
            p0 = j * 16
            v = idx_v[pl.ds(p0, 16)]
            y = v >> 9
            x = v & 511
            toff = (((y >> 3) << 12) + ((x >> 7) << 10)
                    + ((y & 7) << 7) + (x & 127))
            for c in range(3):
                fidx_v[pl.ds(c * SPW + p0, 16)] = toff + ((b * 3 + c) * HW)
            return carry
        lax.fori_loop(lo, hi, body, 0)

    gathers = []
    for q in range(4):
        build(q * (CHUNKS // 4), (q + 1) * (CHUNKS // 4))
        gathers.append(_gather_quarter(table_hbm, fidx_v, gat_v, gsem, q))
    fcp = pltpu.async_copy(fidx_v, fidx_out.at[pl.ds(wid * SPW * 3, SPW * 3)],
                           fsem)
    _pipeline(out_ref, gat_v, stage, rowidx_v, wsem, wid, gathers)
    fcp.wait()


@functools.partial(
    pl.kernel,
    mesh=_MESH,
    out_type=jax.ShapeDtypeStruct((B * N, 16), jnp.float32),
    scratch_types=_SCRATCH,
    compiler_params=_PARAMS,
)
def _sample_origins(fidx_hbm, table_hbm, out_ref,
                    fidx_v, gat_v, stage, rowidx_v, gsem, wsem):
    wid = lax.axis_index("s") * NC + lax.axis_index("c")
    pltpu.sync_copy(fidx_hbm.at[pl.ds(wid * SPW * 3, SPW * 3)], fidx_v)
    gathers = [_gather_quarter(table_hbm, fidx_v, gat_v, gsem, q)
               for q in range(4)]
    _pipeline(out_ref, gat_v, stage, rowidx_v, wsem, wid, gathers)


def kernel(n_sample, rays_directions, rays_origins):
    indices = jax.random.randint(jax.random.key(42), (B, N), 0, HW)
    idx_flat = indices.reshape(-1).astype(jnp.int32)

    def tiled_flat(a):
        return (a.reshape(B, 3, H // 8, 8, W // 128, 128)
                 .transpose(0, 1, 2, 4, 3, 5)
                 .reshape(-1))

    dirs_rows, fidx = _sample_dirs(idx_flat, tiled_flat(rays_directions))
    orig_rows = _sample_origins(fidx, tiled_flat(rays_origins))
    sampled_dirs = dirs_rows.reshape(B, N, 16)[:, :, :3]
    sampled_origins = orig_rows.reshape(B, N, 16)[:, :, :3]

    sample_coordinates = jnp.stack((indices >> 9, indices & 511), axis=-1)
    sample_coordinates = sample_coordinates.astype(jnp.int32)

    indices = indices + (jnp.asarray(n_sample, dtype=indices.dtype) * 0)
    return indices, sample_coordinates, sampled_dirs, sampled_origins


# speedup vs baseline: 1.2250x; 1.2250x over previous
import functools

import jax
import jax.numpy as jnp
from jax import lax
from jax.experimental import pallas as pl
from jax.experimental.pallas import tpu as pltpu
from jax.experimental.pallas import tpu_sc as plsc

H = 512
W = 512
B = 8
N = 4096
HW = H * W

NC = 2
NS = 16
NW = NC * NS
SPW = (B * N) // NW
CHUNKS = SPW // 16
WPB = NW // B
QS = SPW // 4

_MESH = plsc.VectorSubcoreMesh(core_axis_name="c", subcore_axis_name="s")

_SCRATCH = [
    pltpu.VMEM((SPW * 3,), jnp.int32),
    pltpu.VMEM((SPW * 3,), jnp.float32),
    pltpu.VMEM((QS, 3), jnp.float32),
    pltpu.VMEM((QS, 3), jnp.float32),
    pltpu.SemaphoreType.DMA,
    pltpu.SemaphoreType.DMA,
    pltpu.SemaphoreType.DMA,
]

_PARAMS = pltpu.CompilerParams(needs_layout_passes=False)


def _gather_quarter(table_hbm, fidx_v, gat_v, gsem, q):
    return [
        pltpu.async_copy(
            table_hbm.at[fidx_v.at[pl.ds(c * SPW + q * QS, QS)]],
            gat_v.at[pl.ds(c * SPW + q * QS, QS)], gsem)
        for c in range(3)
    ]


def _pipeline(out_ref, gat_v, stages, wsems, b, n0, gathers):
    iota = lax.iota(jnp.int32, 16)

    def fill_quarter(q, stage):
        def body(k, carry):
            rows = k * 16 + iota
            for c in range(3):
                vals = gat_v[pl.ds(c * SPW + q * QS + k * 16, 16)]
                plsc.store_scatter(
                    stage, [rows, jnp.full((16,), c, jnp.int32)], vals)
            return carry
        lax.fori_loop(0, QS // 16, body, 0)

    writes = [None, None]
    for q in range(4):
        if writes[q % 2] is not None:
            writes[q % 2].wait()
        for cp in gathers[q]:
            cp.wait()
        fill_quarter(q, stages[q % 2])
        writes[q % 2] = pltpu.async_copy(
            stages[q % 2], out_ref.at[b, pl.ds(n0 + q * QS, QS)],
            wsems[q % 2])
    writes[0].wait()
    writes[1].wait()


@functools.partial(
    pl.kernel,
    mesh=_MESH,
    out_type=[
        jax.ShapeDtypeStruct((B, N, 3), jnp.float32),
        jax.ShapeDtypeStruct((B * N * 3,), jnp.int32),
    ],
    scratch_types=[pltpu.VMEM((SPW,), jnp.int32), *_SCRATCH,
                   pltpu.SemaphoreType.DMA],
    compiler_params=_PARAMS,
)
def _sample_dirs(idx_hbm, table_hbm, out_ref, fidx_out,
                 idx_v, fidx_v, gat_v, stage0, stage1, gsem, wsem0, wsem1,
                 fsem):
    wid = lax.axis_index("s") * NC + lax.axis_index("c")
    b = wid // WPB
    n0 = (wid % WPB) * SPW
    pltpu.sync_copy(idx_hbm.at[pl.ds(wid * SPW, SPW)], idx_v)

    def build(lo, hi):
        def body(j, carry):
            p0 = j * 16
            v = idx_v[pl.ds(p0, 16)]
            y = v >> 9
            x = v & 511
            toff = (((y >> 3) << 12) + ((x >> 7) << 10)
                    + ((y & 7) << 7) + (x & 127))
            for c in range(3):
                fidx_v[pl.ds(c * SPW + p0, 16)] = toff + ((b * 3 + c) * HW)
            return carry
        lax.fori_loop(lo, hi, body, 0)

    gathers = []
    for q in range(4):
        build(q * (CHUNKS // 4), (q + 1) * (CHUNKS // 4))
        gathers.append(_gather_quarter(table_hbm, fidx_v, gat_v, gsem, q))
    fcp = pltpu.async_copy(fidx_v, fidx_out.at[pl.ds(wid * SPW * 3, SPW * 3)],
                           fsem)
    _pipeline(out_ref, gat_v, (stage0, stage1), (wsem0, wsem1), b, n0,
              gathers)
    fcp.wait()


@functools.partial(
    pl.kernel,
    mesh=_MESH,
    out_type=jax.ShapeDtypeStruct((B, N, 3), jnp.float32),
    scratch_types=_SCRATCH,
    compiler_params=_PARAMS,
)
def _sample_origins(fidx_hbm, table_hbm, out_ref,
                    fidx_v, gat_v, stage0, stage1, gsem, wsem0, wsem1):
    wid = lax.axis_index("s") * NC + lax.axis_index("c")
    b = wid // WPB
    n0 = (wid % WPB) * SPW
    pltpu.sync_copy(fidx_hbm.at[pl.ds(wid * SPW * 3, SPW * 3)], fidx_v)
    gathers = [_gather_quarter(table_hbm, fidx_v, gat_v, gsem, q)
               for q in range(4)]
    _pipeline(out_ref, gat_v, (stage0, stage1), (wsem0, wsem1), b, n0,
              gathers)


def kernel(n_sample, rays_directions, rays_origins):
    indices = jax.random.randint(jax.random.key(42), (B, N), 0, HW)
    idx_flat = indices.reshape(-1).astype(jnp.int32)

    def tiled_flat(a):
        return (a.reshape(B, 3, H // 8, 8, W // 128, 128)
                 .transpose(0, 1, 2, 4, 3, 5)
                 .reshape(-1))

    sampled_dirs, fidx = _sample_dirs(idx_flat, tiled_flat(rays_directions))
    sampled_origins = _sample_origins(fidx, tiled_flat(rays_origins))

    sample_coordinates = jnp.stack((indices >> 9, indices & 511), axis=-1)
    sample_coordinates = sample_coordinates.astype(jnp.int32)

    indices = indices + (jnp.asarray(n_sample, dtype=indices.dtype) * 0)
    return indices, sample_coordinates, sampled_dirs, sampled_origins
